# baseline (speedup 1.0000x reference)
import math

import jax
import jax.numpy as jnp
from jax import lax
from jax.experimental import pallas as pl
from jax.experimental.pallas import tpu as pltpu

_B = 2048
_E = 64
_K = 8
_ROWS = 72
_C = 128
_CAP = int(math.ceil(1.25 * _B / _E))
_CHUNK = 8


def _route_kernel(rf_ref, wr_ref, tok_ref, wsel_ref, aux_ref):
    logits = lax.dot_general(
        rf_ref[...], wr_ref[...],
        dimension_numbers=(((1,), (1,)), ((), ())),
        preferred_element_type=jnp.float32)
    m = jnp.max(logits, axis=1, keepdims=True)
    ex = jnp.exp(logits - m)
    probs = ex / jnp.sum(ex, axis=1, keepdims=True)

    iota = lax.broadcasted_iota(jnp.int32, (_B, _E), 1)
    work = probs
    vals = []
    for _ in range(_K):
        mv = jnp.max(work, axis=1, keepdims=True)
        ik = jnp.min(jnp.where(work == mv, iota, jnp.int32(2 ** 30)),
                     axis=1, keepdims=True)
        vals.append(mv)
        work = jnp.where(iota == ik, jnp.float32(-1.0), work)

    denom = vals[0] + vals[1] + vals[2] + vals[3] + \
        vals[4] + vals[5] + vals[6] + vals[7]
    inv = 1.0 / (denom + 1e-9)
    mask8 = work < -0.5
    cand = jnp.where(mask8, probs * inv, jnp.float32(-1.0))

    lo = jnp.zeros((1, _E), jnp.int32)
    hi = jnp.full((1, _E), 0x3F800001, jnp.int32)
    for _ in range(30):
        mid = lo + ((hi - lo) >> 1)
        midf = lax.bitcast_convert_type(mid, jnp.float32)
        cnt = jnp.sum((cand >= midf).astype(jnp.float32), axis=0,
                      keepdims=True)
        pred = cnt >= jnp.float32(_CAP)
        lo = jnp.where(pred, mid, lo)
        hi = jnp.where(pred, hi, mid)
    tf = lax.bitcast_convert_type(lo, jnp.float32)

    kept = (cand >= tf).astype(jnp.float32)

    s2 = jnp.sum(kept * jnp.maximum(cand, 0.0), axis=1, keepdims=True)
    wf = kept * jnp.maximum(cand, 0.0) / (s2 + 1e-9)

    incl = kept
    sh = 1
    while sh < _B:
        incl = incl + jnp.concatenate(
            [jnp.zeros((sh, _E), jnp.float32), incl[:-sh]], axis=0)
        sh *= 2
    r = incl - kept

    tok_f = lax.broadcasted_iota(jnp.int32, (_B, _E), 0).astype(jnp.float32)
    tok_rows, w_rows = [], []
    for s in range(_CAP):
        m_s = kept * (r == jnp.float32(s)).astype(jnp.float32)
        tok_rows.append(jnp.sum(m_s * tok_f, axis=0, keepdims=True))
        w_rows.append(jnp.sum(m_s * wf, axis=0, keepdims=True))
    tok_ref[...] = jnp.concatenate(tok_rows, axis=0).astype(jnp.int32)
    wsel_ref[...] = jnp.concatenate(w_rows, axis=0)

    imp = jnp.sum(probs, axis=0, keepdims=True)
    load = jnp.sum(kept, axis=0, keepdims=True)
    imp_n = imp / (jnp.sum(imp, keepdims=True) + 1e-9)
    load_n = load / (jnp.sum(load, keepdims=True) + 1e-9)
    aux_ref[...] = jnp.sum(jnp.float32(_E) * imp_n * load_n,
                           keepdims=True)


def _zero_kernel(y_ref):
    y_ref[...] = jnp.zeros_like(y_ref)


def _moe_kernel(tok_ref, w_ref, xh_ref, we_ref, be_ref, yin_ref, yo_ref,
                xbuf, ybuf, semx, semyr, semyw):
    del yin_ref
    e = pl.program_id(0)
    cur = lax.rem(e, 2)
    nxt = 1 - cur

    def issue_x(expert, buf):
        for i in range(_CAP):
            t = tok_ref[i, expert]
            pltpu.make_async_copy(
                xh_ref.at[t], xbuf.at[buf, i], semx.at[buf]).start()

    def wait_x(buf):
        for i in range(_CAP):
            pltpu.make_async_copy(
                xh_ref.at[0], xbuf.at[buf, i], semx.at[buf]).wait()

    def drain_yw(buf):
        for i in range(_CAP):
            pltpu.make_async_copy(
                ybuf.at[buf, i], yo_ref.at[0], semyw.at[buf]).wait()

    @pl.when(e == 0)
    def _():
        issue_x(e, cur)

    @pl.when(e + 1 < _E)
    def _():
        issue_x(e + 1, nxt)

    @pl.when(e > 0)
    def _():
        drain_yw(nxt)

    for i in range(_CAP):
        t = tok_ref[i, e]
        pltpu.make_async_copy(
            yo_ref.at[t], ybuf.at[cur, i], semyr.at[cur]).start()

    wait_x(cur)

    for i in range(_CAP):
        pltpu.make_async_copy(
            yo_ref.at[0], ybuf.at[cur, i], semyr.at[cur]).wait()

    for c in range(_CAP // _CHUNK):
        xc = xbuf[cur, pl.ds(c * _CHUNK, _CHUNK)]
        xr = xc.reshape(_CHUNK * _ROWS, _C)
        h = lax.dot_general(
            xr, we_ref[0],
            dimension_numbers=(((1,), (0,)), ((), ())),
            preferred_element_type=jnp.float32)
        h = h + be_ref[pl.ds(e, 1), :]
        act = h * jax.nn.sigmoid(h)
        for i in range(_CHUNK):
            s = c * _CHUNK + i
            wv = w_ref[s, e]
            ybuf[cur, s] = ybuf[cur, s] + wv * act[i * _ROWS:(i + 1) * _ROWS]

    for i in range(_CAP):
        t = tok_ref[i, e]
        pltpu.make_async_copy(
            ybuf.at[cur, i], yo_ref.at[t], semyw.at[cur]).start()

    @pl.when(e == _E - 1)
    def _():
        drain_yw(cur)


def kernel(node_embedding, router_fea, Wr, We, be):
    B, N, L2, C = node_embedding.shape
    E = Wr.shape[0]

    tok, wsel, aux = pl.pallas_call(
        _route_kernel,
        out_shape=(
            jax.ShapeDtypeStruct((_CAP, E), jnp.int32),
            jax.ShapeDtypeStruct((_CAP, E), jnp.float32),
            jax.ShapeDtypeStruct((1, 1), jnp.float32),
        ),
    )(router_fea, Wr)

    y0 = pl.pallas_call(
        _zero_kernel,
        grid=(16,),
        out_specs=pl.BlockSpec((B // 16, N * L2, C), lambda i: (i, 0, 0)),
        out_shape=jax.ShapeDtypeStruct((B, N * L2, C), jnp.float32),
    )()

    y3 = pl.pallas_call(
        _moe_kernel,
        grid=(E,),
        in_specs=[
            pl.BlockSpec((_CAP, E), lambda e: (0, 0),
                         memory_space=pltpu.MemorySpace.SMEM),
            pl.BlockSpec((_CAP, E), lambda e: (0, 0),
                         memory_space=pltpu.MemorySpace.SMEM),
            pl.BlockSpec(memory_space=pltpu.MemorySpace.HBM),
            pl.BlockSpec((1, C, C), lambda e: (e, 0, 0)),
            pl.BlockSpec((E, C), lambda e: (0, 0)),
            pl.BlockSpec(memory_space=pltpu.MemorySpace.HBM),
        ],
        out_specs=pl.BlockSpec(memory_space=pltpu.MemorySpace.HBM),
        out_shape=jax.ShapeDtypeStruct((B, N * L2, C), jnp.float32),
        scratch_shapes=[
            pltpu.VMEM((2, _CAP, 8, 9, _C), jnp.float32),
            pltpu.VMEM((2, _CAP, _ROWS, _C), jnp.float32),
            pltpu.SemaphoreType.DMA((2,)),
            pltpu.SemaphoreType.DMA((2,)),
            pltpu.SemaphoreType.DMA((2,)),
        ],
        input_output_aliases={5: 0},
    )(tok, wsel, node_embedding, We, be, y0)

    return y3.reshape(B, N, L2, C), aux.reshape(())


# speedup vs baseline: 1.0266x; 1.0266x over previous
import math

import jax
import jax.numpy as jnp
from jax import lax
from jax.experimental import pallas as pl
from jax.experimental.pallas import tpu as pltpu

_B = 2048
_E = 64
_K = 8
_ROWS = 72
_C = 128
_CAP = int(math.ceil(1.25 * _B / _E))
_CHUNK = 20


def _route_kernel(rf_ref, wr_ref, tok_ref, wsel_ref, aux_ref):
    logits = lax.dot_general(
        rf_ref[...], wr_ref[...],
        dimension_numbers=(((1,), (1,)), ((), ())),
        preferred_element_type=jnp.float32)
    m = jnp.max(logits, axis=1, keepdims=True)
    ex = jnp.exp(logits - m)
    probs = ex / jnp.sum(ex, axis=1, keepdims=True)

    iota = lax.broadcasted_iota(jnp.int32, (_B, _E), 1)
    work = probs
    vals = []
    for _ in range(_K):
        mv = jnp.max(work, axis=1, keepdims=True)
        ik = jnp.min(jnp.where(work == mv, iota, jnp.int32(2 ** 30)),
                     axis=1, keepdims=True)
        vals.append(mv)
        work = jnp.where(iota == ik, jnp.float32(-1.0), work)

    denom = vals[0] + vals[1] + vals[2] + vals[3] + \
        vals[4] + vals[5] + vals[6] + vals[7]
    inv = 1.0 / (denom + 1e-9)
    mask8 = work < -0.5
    cand = jnp.where(mask8, probs * inv, jnp.float32(-1.0))

    lo = jnp.zeros((1, _E), jnp.int32)
    hi = jnp.full((1, _E), 0x3F800001, jnp.int32)
    for _ in range(30):
        mid = lo + ((hi - lo) >> 1)
        midf = lax.bitcast_convert_type(mid, jnp.float32)
        cnt = jnp.sum((cand >= midf).astype(jnp.float32), axis=0,
                      keepdims=True)
        pred = cnt >= jnp.float32(_CAP)
        lo = jnp.where(pred, mid, lo)
        hi = jnp.where(pred, hi, mid)
    tf = lax.bitcast_convert_type(lo, jnp.float32)

    kept = (cand >= tf).astype(jnp.float32)

    s2 = jnp.sum(kept * jnp.maximum(cand, 0.0), axis=1, keepdims=True)
    wf = kept * jnp.maximum(cand, 0.0) / (s2 + 1e-9)

    incl = kept
    sh = 1
    while sh < _B:
        incl = incl + jnp.concatenate(
            [jnp.zeros((sh, _E), jnp.float32), incl[:-sh]], axis=0)
        sh *= 2
    r = incl - kept

    tok_f = lax.broadcasted_iota(jnp.int32, (_B, _E), 0).astype(jnp.float32)
    tok_rows, w_rows = [], []
    for s in range(_CAP):
        m_s = kept * (r == jnp.float32(s)).astype(jnp.float32)
        tok_rows.append(jnp.sum(m_s * tok_f, axis=0, keepdims=True))
        w_rows.append(jnp.sum(m_s * wf, axis=0, keepdims=True))
    tok_ref[...] = jnp.concatenate(tok_rows, axis=0).astype(jnp.int32)
    wsel_ref[...] = jnp.concatenate(w_rows, axis=0)

    imp = jnp.sum(probs, axis=0, keepdims=True)
    load = jnp.sum(kept, axis=0, keepdims=True)
    imp_n = imp / (jnp.sum(imp, keepdims=True) + 1e-9)
    load_n = load / (jnp.sum(load, keepdims=True) + 1e-9)
    aux_ref[...] = jnp.sum(jnp.float32(_E) * imp_n * load_n,
                           keepdims=True)


def _zero_kernel(y_ref):
    y_ref[...] = jnp.zeros_like(y_ref)


def _moe_kernel(tok_ref, w_ref, xh_ref, we_ref, be_ref, yin_ref, yo_ref,
                xbuf, ybuf, semx, semyr, semyw):
    del yin_ref
    e = pl.program_id(0)
    cur = lax.rem(e, 2)
    nxt = 1 - cur

    def issue_x(expert, buf):
        for i in range(_CAP):
            t = tok_ref[i, expert]
            pltpu.make_async_copy(
                xh_ref.at[t], xbuf.at[buf, i], semx.at[buf]).start()

    def wait_x(buf):
        for i in range(_CAP):
            pltpu.make_async_copy(
                xh_ref.at[0], xbuf.at[buf, i], semx.at[buf]).wait()

    def drain_yw(buf):
        for i in range(_CAP):
            pltpu.make_async_copy(
                ybuf.at[buf, i], yo_ref.at[0], semyw.at[buf]).wait()

    @pl.when(e == 0)
    def _():
        issue_x(e, cur)

    @pl.when(e + 1 < _E)
    def _():
        issue_x(e + 1, nxt)

    @pl.when(e > 0)
    def _():
        drain_yw(nxt)

    for i in range(_CAP):
        t = tok_ref[i, e]
        pltpu.make_async_copy(
            yo_ref.at[t], ybuf.at[cur, i], semyr.at[cur]).start()

    wait_x(cur)

    for i in range(_CAP):
        pltpu.make_async_copy(
            yo_ref.at[0], ybuf.at[cur, i], semyr.at[cur]).wait()

    for c in range(_CAP // _CHUNK):
        xc = xbuf[cur, pl.ds(c * _CHUNK, _CHUNK)]
        xr = xc.reshape(_CHUNK * _ROWS, _C)
        h = lax.dot_general(
            xr, we_ref[0],
            dimension_numbers=(((1,), (0,)), ((), ())),
            preferred_element_type=jnp.float32)
        h = h + be_ref[pl.ds(e, 1), :]
        act = h * jax.nn.sigmoid(h)
        for i in range(_CHUNK):
            s = c * _CHUNK + i
            wv = w_ref[s, e]
            ybuf[cur, s] = ybuf[cur, s] + wv * act[i * _ROWS:(i + 1) * _ROWS]

    for i in range(_CAP):
        t = tok_ref[i, e]
        pltpu.make_async_copy(
            ybuf.at[cur, i], yo_ref.at[t], semyw.at[cur]).start()

    @pl.when(e == _E - 1)
    def _():
        drain_yw(cur)


def kernel(node_embedding, router_fea, Wr, We, be):
    B, N, L2, C = node_embedding.shape
    E = Wr.shape[0]

    tok, wsel, aux = pl.pallas_call(
        _route_kernel,
        out_shape=(
            jax.ShapeDtypeStruct((_CAP, E), jnp.int32),
            jax.ShapeDtypeStruct((_CAP, E), jnp.float32),
            jax.ShapeDtypeStruct((1, 1), jnp.float32),
        ),
    )(router_fea, Wr)

    y0 = pl.pallas_call(
        _zero_kernel,
        grid=(16,),
        out_specs=pl.BlockSpec((B // 16, N * L2, C), lambda i: (i, 0, 0)),
        out_shape=jax.ShapeDtypeStruct((B, N * L2, C), jnp.float32),
    )()

    x3 = node_embedding.reshape(B, N * L2, C)
    y3 = pl.pallas_call(
        _moe_kernel,
        grid=(E,),
        in_specs=[
            pl.BlockSpec((_CAP, E), lambda e: (0, 0),
                         memory_space=pltpu.MemorySpace.SMEM),
            pl.BlockSpec((_CAP, E), lambda e: (0, 0),
                         memory_space=pltpu.MemorySpace.SMEM),
            pl.BlockSpec(memory_space=pltpu.MemorySpace.HBM),
            pl.BlockSpec((1, C, C), lambda e: (e, 0, 0)),
            pl.BlockSpec((E, C), lambda e: (0, 0)),
            pl.BlockSpec(memory_space=pltpu.MemorySpace.HBM),
        ],
        out_specs=pl.BlockSpec(memory_space=pltpu.MemorySpace.HBM),
        out_shape=jax.ShapeDtypeStruct((B, N * L2, C), jnp.float32),
        scratch_shapes=[
            pltpu.VMEM((2, _CAP, _ROWS, _C), jnp.float32),
            pltpu.VMEM((2, _CAP, _ROWS, _C), jnp.float32),
            pltpu.SemaphoreType.DMA((2,)),
            pltpu.SemaphoreType.DMA((2,)),
            pltpu.SemaphoreType.DMA((2,)),
        ],
        input_output_aliases={5: 0},
    )(tok, wsel, x3, We, be, y0)

    return y3.reshape(B, N, L2, C), aux.reshape(())
